# Initial kernel scaffold; baseline (speedup 1.0000x reference)
import jax
import jax.numpy as jnp
from jax.experimental import pallas as pl


def kernel(x, edge_index, W1, b1, W2, b2, W3, b3, Wl, bl):
    raise NotImplementedError("write your pallas kernel here")


# speedup vs baseline: 8.0217x; 8.0217x over previous
import functools

import jax
import jax.numpy as jnp
from jax import lax
from jax.experimental import pallas as pl
from jax.experimental.pallas import tpu as pltpu
from jax.experimental.pallas import tpu_sc as plsc

N = 10000
E = 320000
D = 128
D2 = D // 2
K = 5

NC = 2
NS = 16
NW = NC * NS
EPT = E // NW
CH = 125
NCH = EPT // CH
NPAIR = NCH // 2

N_PAD = 10240
RPT = N_PAD // NS

def _sc_mesh():
    return plsc.VectorSubcoreMesh(core_axis_name="c", subcore_axis_name="s",
                                  num_cores=NC, num_subcores=NS)


@functools.cache
def _build_deg_sc():
    return functools.partial(
        pl.kernel,
        out_type=jax.ShapeDtypeStruct((NC, N_PAD), jnp.float32),
        mesh=_sc_mesh(),
        scratch_types=[
            pltpu.VMEM((NCH, CH), jnp.int32),
            pltpu.VMEM((128,), jnp.float32),
            pltpu.VMEM((RPT,), jnp.float32),
            pltpu.VMEM_SHARED((N_PAD,), jnp.float32),
        ],
    )(_deg_sc_body)


def _deg_sc(row_r):
    return _build_deg_sc()(row_r)


def _deg_sc_body(row_hbm, out_hbm, rowv, ones_v, zed_v, acc):
    c = lax.axis_index("c")
    s = lax.axis_index("s")
    wid = c * NS + s

    pltpu.sync_copy(row_hbm.at[wid], rowv)

    @pl.loop(0, 8)
    def _fill(i):
        ones_v[pl.ds(i * 16, 16)] = jnp.full((16,), 1.0, jnp.float32)

    @pl.loop(0, RPT // 16)
    def _zed(i):
        zed_v[pl.ds(i * 16, 16)] = jnp.zeros((16,), jnp.float32)

    pltpu.sync_copy(zed_v, acc.at[pl.ds(s * RPT, RPT)])
    plsc.subcore_barrier()

    @pl.loop(0, NCH)
    def _scatter(j):
        pltpu.sync_copy(ones_v.at[pl.ds(0, CH)], acc.at[rowv.at[j]], add=True)

    plsc.subcore_barrier()
    pltpu.sync_copy(acc.at[pl.ds(s * RPT, RPT)],
                    out_hbm.at[c, pl.ds(s * RPT, RPT)])


@functools.cache
def _build_prop_sc():
    return functools.partial(
        pl.kernel,
        out_type=[jax.ShapeDtypeStruct((NC, N_PAD, D2), jnp.float32),
                  jax.ShapeDtypeStruct((NC, N_PAD, D2), jnp.float32)],
        mesh=_sc_mesh(),
        scratch_types=[
            pltpu.VMEM((NCH, CH), jnp.int32),
            pltpu.VMEM((NCH, CH), jnp.int32),
            pltpu.VMEM((CH, D2), jnp.float32),
            pltpu.VMEM((CH, D2), jnp.float32),
            pltpu.VMEM((128, D2), jnp.float32),
            pltpu.VMEM_SHARED((N_PAD, D2), jnp.float32),
            pltpu.SemaphoreType.DMA,
            pltpu.SemaphoreType.DMA,
        ],
        compiler_params=pltpu.CompilerParams(use_tc_tiling_on_sc=False),
    )(_prop_sc_body)


def _prop_sc(u0, u1, col_r, row_r):
    return _build_prop_sc()(u0, u1, col_r, row_r)


def _prop_sc_body(u0_hbm, u1_hbm, col_hbm, row_hbm, out0_hbm, out1_hbm,
                  colv, rowv, g0, g1, zed_v, acc, s0, s1):
    c = lax.axis_index("c")
    s = lax.axis_index("s")
    wid = c * NS + s

    pltpu.sync_copy(col_hbm.at[wid], colv)
    pltpu.sync_copy(row_hbm.at[wid], rowv)

    @pl.loop(0, 128)
    def _zed(i):
        for l in range(D2 // 16):
            zed_v[i, pl.ds(l * 16, 16)] = jnp.zeros((16,), jnp.float32)

    @pl.loop(0, RPT // 128)
    def _zcp(i):
        pltpu.sync_copy(zed_v, acc.at[pl.ds(s * RPT + i * 128, 128)])

    for h, (u_hbm, out_hbm) in enumerate(((u0_hbm, out0_hbm),
                                          (u1_hbm, out1_hbm))):
        plsc.subcore_barrier()

        pltpu.async_copy(u_hbm.at[colv.at[0]], g0, s0)

        @pl.loop(0, NPAIR)
        def _pair(i):
            j0 = i * 2
            pltpu.make_async_copy(u_hbm.at[colv.at[j0]], g0, s0).wait()
            pltpu.async_copy(u_hbm.at[colv.at[j0 + 1]], g1, s1)
            pltpu.sync_copy(g0, acc.at[rowv.at[j0]], add=True)
            pltpu.make_async_copy(u_hbm.at[colv.at[j0 + 1]], g1, s1).wait()

            @pl.when(i + 1 < NPAIR)
            def _():
                pltpu.async_copy(u_hbm.at[colv.at[j0 + 2]], g0, s0)

            pltpu.sync_copy(g1, acc.at[rowv.at[j0 + 1]], add=True)

        plsc.subcore_barrier()
        pltpu.sync_copy(acc.at[pl.ds(s * RPT, RPT)],
                        out_hbm.at[c, pl.ds(s * RPT, RPT)])
        if h == 0:
            @pl.loop(0, RPT // 128)
            def _rz(i):
                pltpu.sync_copy(zed_v, acc.at[pl.ds(s * RPT + i * 128, 128)])


_BR = 512
_GRID = N_PAD // _BR

_row_spec = pl.BlockSpec((_BR, D), lambda i: (i, 0))
_w_spec = pl.BlockSpec((D, D), lambda i: (0, 0))
_uh_spec = pl.BlockSpec((2, _BR, D2), lambda i: (0, i, 0))
_p_spec = pl.BlockSpec((NC, _BR, D2), lambda i: (0, i, 0))

_f32 = jnp.float32


def _split_u(u_ref, u):
    u_ref[0] = u[:, :D2]
    u_ref[1] = u[:, D2:]


def _tc_prep_body(dp_ref, dinvb_ref):
    deg = dp_ref[0] + dp_ref[1]
    dinv = jnp.where(deg > 0.0, lax.rsqrt(jnp.maximum(deg, 1e-12)), 0.0)
    dinvb_ref[...] = jnp.broadcast_to(dinv[:, None], (_BR, D))


_tc_prep = pl.pallas_call(
    _tc_prep_body,
    grid=(_GRID,),
    in_specs=[pl.BlockSpec((NC, _BR), lambda i: (0, i))],
    out_specs=_row_spec,
    out_shape=jax.ShapeDtypeStruct((N_PAD, D), _f32),
)


def _make_tc_start(first):
    def body(*refs):
        if first:
            h_ref, dinvb_ref, w_ref, u_ref, acc_ref = refs
            h = h_ref[...]
        else:
            accin_ref, b_ref, dinvb_ref, w_ref, h_ref, u_ref, acc_ref = refs
            h = jnp.maximum(accin_ref[...] + b_ref[...], 0.0)
            h_ref[...] = h
        _split_u(u_ref, dinvb_ref[...] * h)
        acc_ref[...] = jnp.dot(h, w_ref[...], preferred_element_type=_f32)

    u_shape = jax.ShapeDtypeStruct((2, N_PAD, D2), _f32)
    nd_shape = jax.ShapeDtypeStruct((N_PAD, D), _f32)
    if first:
        in_specs = [_row_spec, _row_spec, _w_spec]
        out_shape = [u_shape, nd_shape]
        out_specs = [_uh_spec, _row_spec]
    else:
        in_specs = [_row_spec, pl.BlockSpec((1, D), lambda i: (0, 0)),
                    _row_spec, _w_spec]
        out_shape = [nd_shape, u_shape, nd_shape]
        out_specs = [_row_spec, _uh_spec, _row_spec]
    return pl.pallas_call(body, grid=(_GRID,), in_specs=in_specs,
                          out_specs=out_specs, out_shape=out_shape)


_tc_start_first = _make_tc_start(True)
_tc_start_next = _make_tc_start(False)


def _make_tc_step(with_prev, emit_u):
    alpha = 2.0 if with_prev else 1.0

    def body(*refs):
        if with_prev:
            p0_ref, p1_ref, txpp_ref, dinvb_ref, w_ref, accin_ref, *outs = refs
        else:
            p0_ref, p1_ref, dinvb_ref, w_ref, accin_ref, *outs = refs
        psum = jnp.concatenate(
            [p0_ref[0] + p0_ref[1], p1_ref[0] + p1_ref[1]], axis=1)
        tx = -alpha * dinvb_ref[...] * psum
        if with_prev:
            tx = tx - txpp_ref[...]
        if emit_u:
            tx_ref, u_ref, acc_ref = outs
            _split_u(u_ref, dinvb_ref[...] * tx)
        else:
            tx_ref, acc_ref = outs
        tx_ref[...] = tx
        acc_ref[...] = accin_ref[...] + jnp.dot(
            tx, w_ref[...], preferred_element_type=_f32)

    u_shape = jax.ShapeDtypeStruct((2, N_PAD, D2), _f32)
    nd_shape = jax.ShapeDtypeStruct((N_PAD, D), _f32)
    in_specs = [_p_spec, _p_spec]
    if with_prev:
        in_specs += [_row_spec]
    in_specs += [_row_spec, _w_spec, _row_spec]
    if emit_u:
        out_shape = [nd_shape, u_shape, nd_shape]
        out_specs = [_row_spec, _uh_spec, _row_spec]
    else:
        out_shape = [nd_shape, nd_shape]
        out_specs = [_row_spec, _row_spec]
    return pl.pallas_call(body, grid=(_GRID,), in_specs=in_specs,
                          out_specs=out_specs, out_shape=out_shape)


_tc_step_k1 = _make_tc_step(False, True)
_tc_step_mid = _make_tc_step(True, True)
_tc_step_last = _make_tc_step(True, False)


def _tc_head_body(acc_ref, b_ref, wl_ref, bl_ref, y_ref):
    h = acc_ref[...] + b_ref[...]
    y_ref[...] = jnp.dot(h, wl_ref[...],
                         preferred_element_type=_f32) + bl_ref[0, 0]


_tc_head = pl.pallas_call(
    _tc_head_body,
    grid=(_GRID,),
    in_specs=[_row_spec, pl.BlockSpec((1, D), lambda i: (0, 0)),
              pl.BlockSpec((D, 1), lambda i: (0, 0)),
              pl.BlockSpec((1, 1), lambda i: (0, 0))],
    out_specs=pl.BlockSpec((_BR, 1), lambda i: (i, 0)),
    out_shape=jax.ShapeDtypeStruct((N_PAD, 1), _f32),
)


def kernel(x, edge_index, W1, b1, W2, b2, W3, b3, Wl, bl):
    row_r = edge_index[0].reshape(NW, NCH, CH)
    col_r = edge_index[1].reshape(NW, NCH, CH)

    xp = jnp.concatenate([x, jnp.zeros((N_PAD - N, D), jnp.float32)], axis=0)

    dp = _deg_sc(row_r)
    dinvb = _tc_prep(dp)

    acc = None
    bprev = None
    for li, (W, b) in enumerate(((W1, b1), (W2, b2), (W3, b3))):
        if li == 0:
            h = xp
            u, acc = _tc_start_first(h, dinvb, W[0])
        else:
            h, u, acc = _tc_start_next(acc, bprev.reshape(1, D), dinvb, W[0])
        txs = [h]
        for k in range(1, K):
            p0, p1 = _prop_sc(u[0], u[1], col_r, row_r)
            if k == 1:
                tx, u, acc = _tc_step_k1(p0, p1, dinvb, W[k], acc)
            elif k < K - 1:
                tx, u, acc = _tc_step_mid(p0, p1, txs[k - 2], dinvb, W[k], acc)
            else:
                tx, acc = _tc_step_last(p0, p1, txs[k - 2], dinvb, W[k], acc)
            txs.append(tx)
        bprev = b

    y = _tc_head(acc, b3.reshape(1, D), Wl, bl.reshape(1, 1))
    return y[:N]


# speedup vs baseline: 11.4497x; 1.4273x over previous
import functools

import jax
import jax.numpy as jnp
from jax import lax
from jax.experimental import pallas as pl
from jax.experimental.pallas import tpu as pltpu
from jax.experimental.pallas import tpu_sc as plsc

N = 10000
E = 320000
D = 128
D2 = D // 2
K = 5

NC = 2
NS = 16
NW = NC * NS
EPT = E // NW
CH = 125
NCH = EPT // CH
NB = 5
LEAD = 3

N_PAD = 10240
RPT = N_PAD // NS

def _sc_mesh():
    return plsc.VectorSubcoreMesh(core_axis_name="c", subcore_axis_name="s",
                                  num_cores=NC, num_subcores=NS)


@functools.cache
def _build_deg_sc():
    return functools.partial(
        pl.kernel,
        out_type=jax.ShapeDtypeStruct((NC, N_PAD), jnp.float32),
        mesh=_sc_mesh(),
        scratch_types=[
            pltpu.VMEM((NCH, CH), jnp.int32),
            pltpu.VMEM((128,), jnp.float32),
            pltpu.VMEM((RPT,), jnp.float32),
            pltpu.VMEM_SHARED((N_PAD,), jnp.float32),
        ],
    )(_deg_sc_body)


def _deg_sc(row_r):
    return _build_deg_sc()(row_r)


def _deg_sc_body(row_hbm, out_hbm, rowv, ones_v, zed_v, acc):
    c = lax.axis_index("c")
    s = lax.axis_index("s")
    wid = c * NS + s

    pltpu.sync_copy(row_hbm.at[wid], rowv)

    @pl.loop(0, 8)
    def _fill(i):
        ones_v[pl.ds(i * 16, 16)] = jnp.full((16,), 1.0, jnp.float32)

    @pl.loop(0, RPT // 16)
    def _zed(i):
        zed_v[pl.ds(i * 16, 16)] = jnp.zeros((16,), jnp.float32)

    pltpu.sync_copy(zed_v, acc.at[pl.ds(s * RPT, RPT)])
    plsc.subcore_barrier()

    @pl.loop(0, NCH)
    def _scatter(j):
        pltpu.sync_copy(ones_v.at[pl.ds(0, CH)], acc.at[rowv.at[j]], add=True)

    plsc.subcore_barrier()
    pltpu.sync_copy(acc.at[pl.ds(s * RPT, RPT)],
                    out_hbm.at[c, pl.ds(s * RPT, RPT)])


@functools.cache
def _build_prop_sc():
    return functools.partial(
        pl.kernel,
        out_type=[jax.ShapeDtypeStruct((NC, N_PAD, D2), jnp.float32),
                  jax.ShapeDtypeStruct((NC, N_PAD, D2), jnp.float32)],
        mesh=_sc_mesh(),
        scratch_types=[
            pltpu.VMEM((NCH, CH), jnp.int32),
            pltpu.VMEM((NCH, CH), jnp.int32),
            [pltpu.VMEM((CH, D2), jnp.float32)] * NB,
            pltpu.VMEM((128, D2), jnp.float32),
            pltpu.VMEM_SHARED((N_PAD, D2), jnp.float32),
            [pltpu.SemaphoreType.DMA] * NB,
            [pltpu.SemaphoreType.DMA] * NB,
        ],
        compiler_params=pltpu.CompilerParams(use_tc_tiling_on_sc=False),
    )(_prop_sc_body)


def _prop_sc(u0, u1, col_r, row_r):
    return _build_prop_sc()(u0, u1, col_r, row_r)


def _prop_sc_body(u0_hbm, u1_hbm, col_hbm, row_hbm, out0_hbm, out1_hbm,
                  colv, rowv, gb, zed_v, acc, gsem, ssem):
    c = lax.axis_index("c")
    s = lax.axis_index("s")
    wid = c * NS + s

    pltpu.sync_copy(col_hbm.at[wid], colv)
    pltpu.sync_copy(row_hbm.at[wid], rowv)

    @pl.loop(0, 128)
    def _zed(i):
        for l in range(D2 // 16):
            zed_v[i, pl.ds(l * 16, 16)] = jnp.zeros((16,), jnp.float32)

    @pl.loop(0, RPT // 128)
    def _zcp(i):
        pltpu.sync_copy(zed_v, acc.at[pl.ds(s * RPT + i * 128, 128)])

    def _gather(j, b, u_hbm):
        return pltpu.async_copy(u_hbm.at[colv.at[j]], gb[b], gsem[b])

    def _scat(j, b):
        return pltpu.async_copy(gb[b], acc.at[rowv.at[j]], ssem[b],
                                add=True)

    ng = NCH // NB
    for h, (u_hbm, out_hbm) in enumerate(((u0_hbm, out0_hbm),
                                          (u1_hbm, out1_hbm))):
        plsc.subcore_barrier()

        for b in range(LEAD):
            _gather(b, b, u_hbm)

        for b in range(NB):
            pltpu.make_async_copy(u_hbm.at[colv.at[b]],
                                  gb[b], gsem[b]).wait()
            _scat(b, b)
            b2 = (b + LEAD) % NB
            if b < NB - LEAD:
                _gather(b + LEAD, b2, u_hbm)
            else:
                pltpu.make_async_copy(
                    gb[b2], acc.at[rowv.at[b - (NB - LEAD)]],
                    ssem[b2]).wait()
                _gather(b + LEAD, b2, u_hbm)

        @pl.loop(1, ng)
        def _grp(g):
            for b in range(NB):
                j = g * NB + b
                pltpu.make_async_copy(u_hbm.at[colv.at[j]],
                                      gb[b], gsem[b]).wait()
                _scat(j, b)

                b2 = (b + LEAD) % NB

                def _refill(j=j, b2=b2):
                    pltpu.make_async_copy(
                        gb[b2], acc.at[rowv.at[j - (NB - LEAD)]],
                        ssem[b2]).wait()
                    _gather(j + LEAD, b2, u_hbm)

                if b < NB - LEAD:
                    _refill()
                else:
                    pl.when(g < ng - 1)(_refill)

        for j in range(NCH - NB, NCH):
            pltpu.make_async_copy(gb[j % NB], acc.at[rowv.at[j]],
                                  ssem[j % NB]).wait()

        plsc.subcore_barrier()
        pltpu.sync_copy(acc.at[pl.ds(s * RPT, RPT)],
                        out_hbm.at[c, pl.ds(s * RPT, RPT)])
        if h == 0:
            @pl.loop(0, RPT // 128)
            def _rz(i):
                pltpu.sync_copy(zed_v, acc.at[pl.ds(s * RPT + i * 128, 128)])


_BR = 512
_GRID = N_PAD // _BR

_row_spec = pl.BlockSpec((_BR, D), lambda i: (i, 0))
_w_spec = pl.BlockSpec((D, D), lambda i: (0, 0))
_uh_spec = pl.BlockSpec((2, _BR, D2), lambda i: (0, i, 0))
_p_spec = pl.BlockSpec((NC, _BR, D2), lambda i: (0, i, 0))

_f32 = jnp.float32


def _split_u(u_ref, u):
    u_ref[0] = u[:, :D2]
    u_ref[1] = u[:, D2:]


def _tc_prep_body(dp_ref, dinvb_ref):
    deg = dp_ref[0] + dp_ref[1]
    dinv = jnp.where(deg > 0.0, lax.rsqrt(jnp.maximum(deg, 1e-12)), 0.0)
    dinvb_ref[...] = jnp.broadcast_to(dinv[:, None], (_BR, D))


_tc_prep = pl.pallas_call(
    _tc_prep_body,
    grid=(_GRID,),
    in_specs=[pl.BlockSpec((NC, _BR), lambda i: (0, i))],
    out_specs=_row_spec,
    out_shape=jax.ShapeDtypeStruct((N_PAD, D), _f32),
)


def _make_tc_start(first):
    def body(*refs):
        if first:
            h_ref, dinvb_ref, w_ref, u_ref, acc_ref = refs
            h = h_ref[...]
        else:
            accin_ref, b_ref, dinvb_ref, w_ref, h_ref, u_ref, acc_ref = refs
            h = jnp.maximum(accin_ref[...] + b_ref[...], 0.0)
            h_ref[...] = h
        _split_u(u_ref, dinvb_ref[...] * h)
        acc_ref[...] = jnp.dot(h, w_ref[...], preferred_element_type=_f32)

    u_shape = jax.ShapeDtypeStruct((2, N_PAD, D2), _f32)
    nd_shape = jax.ShapeDtypeStruct((N_PAD, D), _f32)
    if first:
        in_specs = [_row_spec, _row_spec, _w_spec]
        out_shape = [u_shape, nd_shape]
        out_specs = [_uh_spec, _row_spec]
    else:
        in_specs = [_row_spec, pl.BlockSpec((1, D), lambda i: (0, 0)),
                    _row_spec, _w_spec]
        out_shape = [nd_shape, u_shape, nd_shape]
        out_specs = [_row_spec, _uh_spec, _row_spec]
    return pl.pallas_call(body, grid=(_GRID,), in_specs=in_specs,
                          out_specs=out_specs, out_shape=out_shape)


_tc_start_first = _make_tc_start(True)
_tc_start_next = _make_tc_start(False)


def _make_tc_step(with_prev, emit_u):
    alpha = 2.0 if with_prev else 1.0

    def body(*refs):
        if with_prev:
            p0_ref, p1_ref, txpp_ref, dinvb_ref, w_ref, accin_ref, *outs = refs
        else:
            p0_ref, p1_ref, dinvb_ref, w_ref, accin_ref, *outs = refs
        psum = jnp.concatenate(
            [p0_ref[0] + p0_ref[1], p1_ref[0] + p1_ref[1]], axis=1)
        tx = -alpha * dinvb_ref[...] * psum
        if with_prev:
            tx = tx - txpp_ref[...]
        if emit_u:
            tx_ref, u_ref, acc_ref = outs
            _split_u(u_ref, dinvb_ref[...] * tx)
        else:
            tx_ref, acc_ref = outs
        tx_ref[...] = tx
        acc_ref[...] = accin_ref[...] + jnp.dot(
            tx, w_ref[...], preferred_element_type=_f32)

    u_shape = jax.ShapeDtypeStruct((2, N_PAD, D2), _f32)
    nd_shape = jax.ShapeDtypeStruct((N_PAD, D), _f32)
    in_specs = [_p_spec, _p_spec]
    if with_prev:
        in_specs += [_row_spec]
    in_specs += [_row_spec, _w_spec, _row_spec]
    if emit_u:
        out_shape = [nd_shape, u_shape, nd_shape]
        out_specs = [_row_spec, _uh_spec, _row_spec]
    else:
        out_shape = [nd_shape, nd_shape]
        out_specs = [_row_spec, _row_spec]
    return pl.pallas_call(body, grid=(_GRID,), in_specs=in_specs,
                          out_specs=out_specs, out_shape=out_shape)


_tc_step_k1 = _make_tc_step(False, True)
_tc_step_mid = _make_tc_step(True, True)
_tc_step_last = _make_tc_step(True, False)


def _tc_head_body(acc_ref, b_ref, wl_ref, bl_ref, y_ref):
    h = acc_ref[...] + b_ref[...]
    y_ref[...] = jnp.dot(h, wl_ref[...],
                         preferred_element_type=_f32) + bl_ref[0, 0]


_tc_head = pl.pallas_call(
    _tc_head_body,
    grid=(_GRID,),
    in_specs=[_row_spec, pl.BlockSpec((1, D), lambda i: (0, 0)),
              pl.BlockSpec((D, 1), lambda i: (0, 0)),
              pl.BlockSpec((1, 1), lambda i: (0, 0))],
    out_specs=pl.BlockSpec((_BR, 1), lambda i: (i, 0)),
    out_shape=jax.ShapeDtypeStruct((N_PAD, 1), _f32),
)


def kernel(x, edge_index, W1, b1, W2, b2, W3, b3, Wl, bl):
    row_r = edge_index[0].reshape(NW, NCH, CH)
    col_r = edge_index[1].reshape(NW, NCH, CH)

    xp = jnp.concatenate([x, jnp.zeros((N_PAD - N, D), jnp.float32)], axis=0)

    dp = _deg_sc(row_r)
    dinvb = _tc_prep(dp)

    acc = None
    bprev = None
    for li, (W, b) in enumerate(((W1, b1), (W2, b2), (W3, b3))):
        if li == 0:
            h = xp
            u, acc = _tc_start_first(h, dinvb, W[0])
        else:
            h, u, acc = _tc_start_next(acc, bprev.reshape(1, D), dinvb, W[0])
        txs = [h]
        for k in range(1, K):
            p0, p1 = _prop_sc(u[0], u[1], col_r, row_r)
            if k == 1:
                tx, u, acc = _tc_step_k1(p0, p1, dinvb, W[k], acc)
            elif k < K - 1:
                tx, u, acc = _tc_step_mid(p0, p1, txs[k - 2], dinvb, W[k], acc)
            else:
                tx, acc = _tc_step_last(p0, p1, txs[k - 2], dinvb, W[k], acc)
            txs.append(tx)
        bprev = b

    y = _tc_head(acc, b3.reshape(1, D), Wl, bl.reshape(1, 1))
    return y[:N]


# speedup vs baseline: 11.9026x; 1.0395x over previous
import functools

import jax
import jax.numpy as jnp
from jax import lax
from jax.experimental import pallas as pl
from jax.experimental.pallas import tpu as pltpu
from jax.experimental.pallas import tpu_sc as plsc

N = 10000
E = 320000
D = 128
D2 = D // 2
K = 5

NC = 2
NS = 16
NW = NC * NS
EPT = E // NW
CH = 125
NCH = EPT // CH
NB = 5
LEAD = 3

N_PAD = 10240
RPT = N_PAD // NS

def _sc_mesh():
    return plsc.VectorSubcoreMesh(core_axis_name="c", subcore_axis_name="s",
                                  num_cores=NC, num_subcores=NS)


@functools.cache
def _build_deg_sc():
    return functools.partial(
        pl.kernel,
        out_type=jax.ShapeDtypeStruct((NC, N_PAD), jnp.float32),
        mesh=_sc_mesh(),
        scratch_types=[
            pltpu.VMEM((NCH, CH), jnp.int32),
            pltpu.VMEM((128,), jnp.float32),
            pltpu.VMEM((RPT,), jnp.float32),
            pltpu.VMEM_SHARED((N_PAD,), jnp.float32),
        ],
    )(_deg_sc_body)


def _deg_sc(row_r):
    return _build_deg_sc()(row_r)


def _deg_sc_body(row_hbm, out_hbm, rowv, ones_v, zed_v, acc):
    c = lax.axis_index("c")
    s = lax.axis_index("s")
    wid = c * NS + s

    pltpu.sync_copy(row_hbm.at[wid], rowv)

    @pl.loop(0, 8)
    def _fill(i):
        ones_v[pl.ds(i * 16, 16)] = jnp.full((16,), 1.0, jnp.float32)

    @pl.loop(0, RPT // 16)
    def _zed(i):
        zed_v[pl.ds(i * 16, 16)] = jnp.zeros((16,), jnp.float32)

    pltpu.sync_copy(zed_v, acc.at[pl.ds(s * RPT, RPT)])
    plsc.subcore_barrier()

    @pl.loop(0, NCH)
    def _scatter(j):
        pltpu.sync_copy(ones_v.at[pl.ds(0, CH)], acc.at[rowv.at[j]], add=True)

    plsc.subcore_barrier()
    pltpu.sync_copy(acc.at[pl.ds(s * RPT, RPT)],
                    out_hbm.at[c, pl.ds(s * RPT, RPT)])


@functools.cache
def _build_prop_sc():
    return functools.partial(
        pl.kernel,
        out_type=[jax.ShapeDtypeStruct((N_PAD, D2), jnp.float32)] * 4,
        mesh=_sc_mesh(),
        scratch_types=[
            pltpu.VMEM((NCH, CH), jnp.int32),
            pltpu.VMEM((NCH, CH), jnp.int32),
            [pltpu.VMEM((CH, D2), jnp.float32)] * NB,
            pltpu.VMEM((128, D2), jnp.float32),
            pltpu.VMEM_SHARED((N_PAD, D2), jnp.float32),
            [pltpu.SemaphoreType.DMA] * NB,
            [pltpu.SemaphoreType.DMA] * NB,
        ],
        compiler_params=pltpu.CompilerParams(use_tc_tiling_on_sc=False),
    )(_prop_sc_body)


def _prop_sc(u0, u1, col_r, row_r):
    return _build_prop_sc()(u0, u1, col_r, row_r)


def _prop_sc_body(u0_hbm, u1_hbm, col_hbm, row_hbm,
                  o00_hbm, o01_hbm, o10_hbm, o11_hbm,
                  colv, rowv, gb, zed_v, acc, gsem, ssem):
    c = lax.axis_index("c")
    s = lax.axis_index("s")
    wid = c * NS + s

    pltpu.sync_copy(col_hbm.at[wid], colv)
    pltpu.sync_copy(row_hbm.at[wid], rowv)

    @pl.loop(0, 128)
    def _zed(i):
        for l in range(D2 // 16):
            zed_v[i, pl.ds(l * 16, 16)] = jnp.zeros((16,), jnp.float32)

    @pl.loop(0, RPT // 128)
    def _zcp(i):
        pltpu.sync_copy(zed_v, acc.at[pl.ds(s * RPT + i * 128, 128)])

    def _gather(j, b, u_hbm):
        return pltpu.async_copy(u_hbm.at[colv.at[j]], gb[b], gsem[b])

    def _scat(j, b):
        return pltpu.async_copy(gb[b], acc.at[rowv.at[j]], ssem[b],
                                add=True)

    ng = NCH // NB
    for h, (u_hbm, out_c0, out_c1) in enumerate(((u0_hbm, o00_hbm, o10_hbm),
                                                 (u1_hbm, o01_hbm, o11_hbm))):
        plsc.subcore_barrier()

        for b in range(LEAD):
            _gather(b, b, u_hbm)

        for b in range(NB):
            pltpu.make_async_copy(u_hbm.at[colv.at[b]],
                                  gb[b], gsem[b]).wait()
            _scat(b, b)
            b2 = (b + LEAD) % NB
            if b < NB - LEAD:
                _gather(b + LEAD, b2, u_hbm)
            else:
                pltpu.make_async_copy(
                    gb[b2], acc.at[rowv.at[b - (NB - LEAD)]],
                    ssem[b2]).wait()
                _gather(b + LEAD, b2, u_hbm)

        @pl.loop(1, ng)
        def _grp(g):
            for b in range(NB):
                j = g * NB + b
                pltpu.make_async_copy(u_hbm.at[colv.at[j]],
                                      gb[b], gsem[b]).wait()
                _scat(j, b)

                b2 = (b + LEAD) % NB

                def _refill(j=j, b2=b2):
                    pltpu.make_async_copy(
                        gb[b2], acc.at[rowv.at[j - (NB - LEAD)]],
                        ssem[b2]).wait()
                    _gather(j + LEAD, b2, u_hbm)

                if b < NB - LEAD:
                    _refill()
                else:
                    pl.when(g < ng - 1)(_refill)

        for j in range(NCH - NB, NCH):
            pltpu.make_async_copy(gb[j % NB], acc.at[rowv.at[j]],
                                  ssem[j % NB]).wait()

        plsc.subcore_barrier()

        @pl.when(c == 0)
        def _wr0():
            pltpu.sync_copy(acc.at[pl.ds(s * RPT, RPT)],
                            out_c0.at[pl.ds(s * RPT, RPT)])

        @pl.when(c == 1)
        def _wr1():
            pltpu.sync_copy(acc.at[pl.ds(s * RPT, RPT)],
                            out_c1.at[pl.ds(s * RPT, RPT)])

        if h == 0:
            @pl.loop(0, RPT // 128)
            def _rz(i):
                pltpu.sync_copy(zed_v, acc.at[pl.ds(s * RPT + i * 128, 128)])


_BR = 512
_GRID = N_PAD // _BR

_row_spec = pl.BlockSpec((_BR, D), lambda i: (i, 0))
_w_spec = pl.BlockSpec((D, D), lambda i: (0, 0))
_half_spec = pl.BlockSpec((_BR, D2), lambda i: (i, 0))

_f32 = jnp.float32


def _split_u(u0_ref, u1_ref, u):
    u0_ref[...] = u[:, :D2]
    u1_ref[...] = u[:, D2:]


def _tc_prep_body(dp_ref, dinvb_ref):
    deg = dp_ref[0] + dp_ref[1]
    dinv = jnp.where(deg > 0.0, lax.rsqrt(jnp.maximum(deg, 1e-12)), 0.0)
    dinvb_ref[...] = jnp.broadcast_to(dinv[:, None], (_BR, D))


_tc_prep = pl.pallas_call(
    _tc_prep_body,
    grid=(_GRID,),
    in_specs=[pl.BlockSpec((NC, _BR), lambda i: (0, i))],
    out_specs=_row_spec,
    out_shape=jax.ShapeDtypeStruct((N_PAD, D), _f32),
)


def _make_tc_start(first):
    def body(*refs):
        if first:
            h_ref, dinvb_ref, w_ref, u0_ref, u1_ref, acc_ref = refs
            h = h_ref[...]
        else:
            (accin_ref, b_ref, dinvb_ref, w_ref,
             h_ref, u0_ref, u1_ref, acc_ref) = refs
            h = jnp.maximum(accin_ref[...] + b_ref[...], 0.0)
            h_ref[...] = h
        _split_u(u0_ref, u1_ref, dinvb_ref[...] * h)
        acc_ref[...] = jnp.dot(h, w_ref[...], preferred_element_type=_f32)

    u_shape = jax.ShapeDtypeStruct((N_PAD, D2), _f32)
    nd_shape = jax.ShapeDtypeStruct((N_PAD, D), _f32)
    if first:
        in_specs = [_row_spec, _row_spec, _w_spec]
        out_shape = [u_shape, u_shape, nd_shape]
        out_specs = [_half_spec, _half_spec, _row_spec]
    else:
        in_specs = [_row_spec, pl.BlockSpec((1, D), lambda i: (0, 0)),
                    _row_spec, _w_spec]
        out_shape = [nd_shape, u_shape, u_shape, nd_shape]
        out_specs = [_row_spec, _half_spec, _half_spec, _row_spec]
    return pl.pallas_call(body, grid=(_GRID,), in_specs=in_specs,
                          out_specs=out_specs, out_shape=out_shape)


_tc_start_first = _make_tc_start(True)
_tc_start_next = _make_tc_start(False)


def _make_tc_step(with_prev, emit_u):
    alpha = 2.0 if with_prev else 1.0

    def body(*refs):
        if with_prev:
            (p00_ref, p01_ref, p10_ref, p11_ref, txpp_ref,
             dinvb_ref, w_ref, accin_ref, *outs) = refs
        else:
            (p00_ref, p01_ref, p10_ref, p11_ref,
             dinvb_ref, w_ref, accin_ref, *outs) = refs
        psum = jnp.concatenate(
            [p00_ref[...] + p10_ref[...], p01_ref[...] + p11_ref[...]],
            axis=1)
        tx = -alpha * dinvb_ref[...] * psum
        if with_prev:
            tx = tx - txpp_ref[...]
        if emit_u:
            tx_ref, u0_ref, u1_ref, acc_ref = outs
            _split_u(u0_ref, u1_ref, dinvb_ref[...] * tx)
        else:
            tx_ref, acc_ref = outs
        tx_ref[...] = tx
        acc_ref[...] = accin_ref[...] + jnp.dot(
            tx, w_ref[...], preferred_element_type=_f32)

    u_shape = jax.ShapeDtypeStruct((N_PAD, D2), _f32)
    nd_shape = jax.ShapeDtypeStruct((N_PAD, D), _f32)
    in_specs = [_half_spec] * 4
    if with_prev:
        in_specs += [_row_spec]
    in_specs += [_row_spec, _w_spec, _row_spec]
    if emit_u:
        out_shape = [nd_shape, u_shape, u_shape, nd_shape]
        out_specs = [_row_spec, _half_spec, _half_spec, _row_spec]
    else:
        out_shape = [nd_shape, nd_shape]
        out_specs = [_row_spec, _row_spec]
    return pl.pallas_call(body, grid=(_GRID,), in_specs=in_specs,
                          out_specs=out_specs, out_shape=out_shape)


_tc_step_k1 = _make_tc_step(False, True)
_tc_step_mid = _make_tc_step(True, True)
_tc_step_last = _make_tc_step(True, False)


def _tc_head_body(acc_ref, b_ref, wl_ref, bl_ref, y_ref):
    h = acc_ref[...] + b_ref[...]
    y_ref[...] = jnp.dot(h, wl_ref[...],
                         preferred_element_type=_f32) + bl_ref[0, 0]


_tc_head = pl.pallas_call(
    _tc_head_body,
    grid=(_GRID,),
    in_specs=[_row_spec, pl.BlockSpec((1, D), lambda i: (0, 0)),
              pl.BlockSpec((D, 1), lambda i: (0, 0)),
              pl.BlockSpec((1, 1), lambda i: (0, 0))],
    out_specs=pl.BlockSpec((_BR, 1), lambda i: (i, 0)),
    out_shape=jax.ShapeDtypeStruct((N_PAD, 1), _f32),
)


def kernel(x, edge_index, W1, b1, W2, b2, W3, b3, Wl, bl):
    row_r = edge_index[0].reshape(NW, NCH, CH)
    col_r = edge_index[1].reshape(NW, NCH, CH)

    xp = jnp.concatenate([x, jnp.zeros((N_PAD - N, D), jnp.float32)], axis=0)

    dp = _deg_sc(row_r)
    dinvb = _tc_prep(dp)

    acc = None
    bprev = None
    for li, (W, b) in enumerate(((W1, b1), (W2, b2), (W3, b3))):
        if li == 0:
            h = xp
            u0, u1, acc = _tc_start_first(h, dinvb, W[0])
        else:
            h, u0, u1, acc = _tc_start_next(acc, bprev.reshape(1, D),
                                            dinvb, W[0])
        txs = [h]
        for k in range(1, K):
            ps = _prop_sc(u0, u1, col_r, row_r)
            if k == 1:
                tx, u0, u1, acc = _tc_step_k1(*ps, dinvb, W[k], acc)
            elif k < K - 1:
                tx, u0, u1, acc = _tc_step_mid(*ps, txs[k - 2], dinvb,
                                               W[k], acc)
            else:
                tx, acc = _tc_step_last(*ps, txs[k - 2], dinvb, W[k], acc)
            txs.append(tx)
        bprev = b

    y = _tc_head(acc, b3.reshape(1, D), Wl, bl.reshape(1, 1))
    return y[:N]
